# baseline (speedup 1.0000x reference)
import functools

import jax
import jax.numpy as jnp
from jax.experimental import pallas as pl
from jax.experimental.pallas import tpu as pltpu

_BLK = 2048


def _pass_kernel(n_cols, actions_ref, logits_ref, lp_ref, mode_ref,
                 m_ref, s_ref, g_ref, am_ref):
    j = pl.program_id(0)
    nb = pl.num_programs(0)

    x = logits_ref[...]
    col = j * _BLK + jax.lax.broadcasted_iota(jnp.int32, x.shape, 1)
    x = jnp.where(col < n_cols, x, -jnp.inf)

    @pl.when(j == 0)
    def _init():
        m_ref[...] = jnp.full_like(m_ref, -jnp.inf)
        s_ref[...] = jnp.zeros_like(s_ref)
        g_ref[...] = jnp.zeros_like(g_ref)
        am_ref[...] = jnp.zeros_like(am_ref)

    run_m = m_ref[...]
    bm = jnp.max(x, axis=1, keepdims=True)
    new_m = jnp.maximum(run_m, bm)

    s_ref[...] = (s_ref[...] * jnp.exp(run_m - new_m)
                  + jnp.sum(jnp.exp(x - new_m), axis=1, keepdims=True))

    big = jnp.int32(2 ** 30)
    cand = jnp.where(x == bm, col, big)
    barg = jnp.min(cand, axis=1, keepdims=True)
    am_ref[...] = jnp.where(bm > run_m, barg, am_ref[...])
    m_ref[...] = new_m

    a = actions_ref[...]
    g_ref[...] += jnp.sum(jnp.where(col == a, x, 0.0), axis=1, keepdims=True)

    @pl.when(j == nb - 1)
    def _fin():
        lp_ref[...] = g_ref[...] - (jnp.log(s_ref[...]) + m_ref[...])
        mode_ref[...] = am_ref[...]


@jax.jit
def kernel(logits, actions):
    b, n = logits.shape
    nb = pl.cdiv(n, _BLK)
    actions = actions.astype(jnp.int32)
    lp, mode = pl.pallas_call(
        functools.partial(_pass_kernel, n),
        grid=(nb,),
        in_specs=[
            pl.BlockSpec((b, 1), lambda j: (0, 0)),
            pl.BlockSpec((b, _BLK), lambda j: (0, j)),
        ],
        out_specs=[
            pl.BlockSpec((b, 1), lambda j: (0, 0)),
            pl.BlockSpec((b, 1), lambda j: (0, 0)),
        ],
        out_shape=[
            jax.ShapeDtypeStruct((b, 1), jnp.float32),
            jax.ShapeDtypeStruct((b, 1), jnp.int32),
        ],
        scratch_shapes=[
            pltpu.VMEM((b, 1), jnp.float32),
            pltpu.VMEM((b, 1), jnp.float32),
            pltpu.VMEM((b, 1), jnp.float32),
            pltpu.VMEM((b, 1), jnp.int32),
        ],
    )(actions, logits)
    return lp, mode


# speedup vs baseline: 1.1993x; 1.1993x over previous
import functools

import jax
import jax.numpy as jnp
from jax.experimental import pallas as pl
from jax.experimental.pallas import tpu as pltpu

_BLK = 8192


def _pass_kernel(n_cols, actions_ref, logits_ref, lp_ref, mode_ref,
                 m_ref, s_ref, g_ref, am_ref):
    j = pl.program_id(0)
    nb = pl.num_programs(0)

    x = logits_ref[...]
    col = j * _BLK + jax.lax.broadcasted_iota(jnp.int32, x.shape, 1)
    x = jnp.where(col < n_cols, x, -jnp.inf)

    @pl.when(j == 0)
    def _init():
        m_ref[...] = jnp.full_like(m_ref, -jnp.inf)
        s_ref[...] = jnp.zeros_like(s_ref)
        g_ref[...] = jnp.zeros_like(g_ref)
        am_ref[...] = jnp.zeros_like(am_ref)

    run_m = m_ref[...]
    bm = jnp.max(x, axis=1, keepdims=True)
    new_m = jnp.maximum(run_m, bm)

    s_ref[...] = (s_ref[...] * jnp.exp(run_m - new_m)
                  + jnp.sum(jnp.exp(x - new_m), axis=1, keepdims=True))

    big = jnp.int32(2 ** 30)
    cand = jnp.where(x == bm, col, big)
    barg = jnp.min(cand, axis=1, keepdims=True)
    am_ref[...] = jnp.where(bm > run_m, barg, am_ref[...])
    m_ref[...] = new_m

    a = actions_ref[...]
    g_ref[...] += jnp.sum(jnp.where(col == a, x, 0.0), axis=1, keepdims=True)

    @pl.when(j == nb - 1)
    def _fin():
        lp_ref[...] = g_ref[...] - (jnp.log(s_ref[...]) + m_ref[...])
        mode_ref[...] = am_ref[...]


@jax.jit
def kernel(logits, actions):
    b, n = logits.shape
    nb = pl.cdiv(n, _BLK)
    actions = actions.astype(jnp.int32)
    lp, mode = pl.pallas_call(
        functools.partial(_pass_kernel, n),
        grid=(nb,),
        in_specs=[
            pl.BlockSpec((b, 1), lambda j: (0, 0)),
            pl.BlockSpec((b, _BLK), lambda j: (0, j)),
        ],
        out_specs=[
            pl.BlockSpec((b, 1), lambda j: (0, 0)),
            pl.BlockSpec((b, 1), lambda j: (0, 0)),
        ],
        out_shape=[
            jax.ShapeDtypeStruct((b, 1), jnp.float32),
            jax.ShapeDtypeStruct((b, 1), jnp.int32),
        ],
        scratch_shapes=[
            pltpu.VMEM((b, 1), jnp.float32),
            pltpu.VMEM((b, 1), jnp.float32),
            pltpu.VMEM((b, 1), jnp.float32),
            pltpu.VMEM((b, 1), jnp.int32),
        ],
    )(actions, logits)
    return lp, mode
